# baseline (speedup 1.0000x reference)
import functools

import jax
import jax.numpy as jnp
from jax import lax
from jax.experimental import pallas as pl
from jax.experimental.pallas import tpu as pltpu
from jax.experimental.pallas import tpu_sc as plsc

VOCAB = 1000000
EMBED_DIM = 64
BATCH = 16384
HIST_LEN = 50

NC = 2
NS = 16
NW = NC * NS

BBLK = 128
NBJ = BATCH // BBLK
NBLOCKS = HIST_LEN * NBJ
BLOCKS_PER_W = NBLOCKS // NW

T_STRIDE = EMBED_DIM * BATCH
CI_STRIDE = 8 * BATCH
BJ_STRIDE = 8 * BBLK
OUT_ELEMS = HIST_LEN * EMBED_DIM * BATCH

_mesh = plsc.VectorSubcoreMesh(core_axis_name="c", subcore_axis_name="s")


@functools.partial(
    pl.kernel,
    mesh=_mesh,
    compiler_params=pltpu.CompilerParams(
        use_tc_tiling_on_sc=False, needs_layout_passes=False
    ),
    out_type=jax.ShapeDtypeStruct((OUT_ELEMS,), jnp.float32),
    scratch_types=[
        pltpu.VMEM((BBLK,), jnp.int32),
        pltpu.VMEM((BBLK,), jnp.int32),
        pltpu.VMEM((BBLK, EMBED_DIM), jnp.float32),
        pltpu.VMEM((BBLK, EMBED_DIM), jnp.float32),
        pltpu.VMEM((EMBED_DIM * BBLK,), jnp.float32),
        pltpu.VMEM((EMBED_DIM * BBLK,), jnp.float32),
        pltpu.SemaphoreType.DMA((2,)),
        pltpu.SemaphoreType.DMA((2,)),
        pltpu.SemaphoreType.DMA((2,)),
    ],
)
def _embed_kernel(
    xt_hbm, table_hbm, out_hbm,
    ibuf0, ibuf1, rows0, rows1, blk0, blk1, isem, gsem, wsem,
):
    ibuf = (ibuf0, ibuf1)
    rows = (rows0, rows1)
    blk = (blk0, blk1)

    wid = lax.axis_index("s") * NC + lax.axis_index("c")
    n0 = wid * BLOCKS_PER_W

    iota = lax.iota(jnp.int32, 16)
    kvecs = [iota + (16 * k) for k in range(8)]

    def idx_off(n):
        g = n0 + n
        t = g // NBJ
        bj = g % NBJ
        return t * BATCH + bj * BBLK

    def out_base(n):
        g = n0 + n
        t = g // NBJ
        bj = g % NBJ
        return t * T_STRIDE + bj * BJ_STRIDE

    def fire_idx(n, b):
        pltpu.async_copy(
            xt_hbm.at[pl.ds(idx_off(n), BBLK)], ibuf[b], isem.at[b]
        )

    def wait_idx(n, b):
        pltpu.make_async_copy(
            xt_hbm.at[pl.ds(idx_off(n), BBLK)], ibuf[b], isem.at[b]
        ).wait()

    def fire_gather(b):
        pltpu.async_copy(table_hbm.at[ibuf[b]], rows[b], gsem.at[b])

    def wait_gather(b):
        pltpu.make_async_copy(
            table_hbm.at[ibuf[b]], rows[b], gsem.at[b]
        ).wait()

    def transpose(b):
        rows_b = rows[b]
        blk_b = blk[b]

        def cstep(c, carry):
            csplat = lax.broadcast(c, (16,))
            for k in range(8):
                v = plsc.load_gather(rows_b, [kvecs[k], csplat])
                blk_b[pl.ds(c * BBLK + k * 16, 16)] = v
            return carry

        lax.fori_loop(0, EMBED_DIM, cstep, 0)

    def fire_write(n, b):
        base = out_base(n)
        for ci in range(8):
            pltpu.async_copy(
                blk[b].at[pl.ds(ci * 1024, 1024)],
                out_hbm.at[pl.ds(base + ci * CI_STRIDE, 1024)],
                wsem.at[b],
            )

    def wait_write(n, b):
        base = out_base(n)
        for ci in range(8):
            pltpu.make_async_copy(
                blk[b].at[pl.ds(ci * 1024, 1024)],
                out_hbm.at[pl.ds(base + ci * CI_STRIDE, 1024)],
                wsem.at[b],
            ).wait()

    fire_idx(0, 0)
    fire_idx(1, 1)
    wait_idx(0, 0)
    fire_gather(0)

    wait_gather(0)
    transpose(0)
    fire_write(0, 0)
    fire_idx(2, 0)
    wait_idx(1, 1)
    fire_gather(1)

    wait_gather(1)
    transpose(1)
    fire_write(1, 1)
    fire_idx(3, 1)
    wait_idx(2, 0)
    fire_gather(0)

    def pair(g, carry):
        for b in range(2):
            n = g * 2 + b
            wait_gather(b)
            wait_write(n - 2, b)
            transpose(b)
            fire_write(n, b)
            fire_idx(n + 2, b)
            wait_idx(n + 1, 1 - b)
            fire_gather(1 - b)
        return carry

    lax.fori_loop(1, BLOCKS_PER_W // 2 - 1, pair, 0)

    n = BLOCKS_PER_W - 2
    wait_gather(0)
    wait_write(n - 2, 0)
    transpose(0)
    fire_write(n, 0)
    wait_idx(n + 1, 1)
    fire_gather(1)

    n = BLOCKS_PER_W - 1
    wait_gather(1)
    wait_write(n - 2, 1)
    transpose(1)
    fire_write(n, 1)

    wait_write(BLOCKS_PER_W - 2, 0)
    wait_write(BLOCKS_PER_W - 1, 1)


def kernel(x, word_table):
    xt = jnp.swapaxes(x, 0, 1).astype(jnp.int32).reshape(-1)
    flat = _embed_kernel(xt, word_table)
    out = (
        flat.reshape(HIST_LEN, 8, NBJ, 8, BBLK)
        .transpose(0, 1, 3, 2, 4)
        .reshape(HIST_LEN, EMBED_DIM, BATCH)
        .transpose(2, 0, 1)
    )
    return out


# speedup vs baseline: 1.3370x; 1.3370x over previous
import functools

import jax
import jax.numpy as jnp
from jax import lax
from jax.experimental import pallas as pl
from jax.experimental.pallas import tpu as pltpu
from jax.experimental.pallas import tpu_sc as plsc

VOCAB = 1000000
EMBED_DIM = 64
BATCH = 16384
HIST_LEN = 50

NC = 2
NS = 16
NW = NC * NS

BBLK = 128
NBJ = BATCH // BBLK
NBLOCKS = HIST_LEN * NBJ
BLOCKS_PER_W = NBLOCKS // NW

T_STRIDE = EMBED_DIM * BATCH
CI_STRIDE = 8 * BATCH
BJ_STRIDE = 8 * BBLK
OUT_ELEMS = HIST_LEN * EMBED_DIM * BATCH

_mesh = plsc.VectorSubcoreMesh(core_axis_name="c", subcore_axis_name="s")


@functools.partial(
    pl.kernel,
    mesh=_mesh,
    compiler_params=pltpu.CompilerParams(
        use_tc_tiling_on_sc=False, needs_layout_passes=False
    ),
    out_type=jax.ShapeDtypeStruct((OUT_ELEMS,), jnp.float32),
    scratch_types=[
        pltpu.VMEM((BBLK,), jnp.int32),
        pltpu.VMEM((BBLK,), jnp.int32),
        pltpu.VMEM((BBLK, EMBED_DIM), jnp.float32),
        pltpu.VMEM((BBLK, EMBED_DIM), jnp.float32),
        pltpu.VMEM((EMBED_DIM * BBLK,), jnp.float32),
        pltpu.VMEM((EMBED_DIM * BBLK,), jnp.float32),
        pltpu.VMEM((16 * 17,), jnp.float32),
        pltpu.SemaphoreType.DMA((2,)),
        pltpu.SemaphoreType.DMA((2,)),
        pltpu.SemaphoreType.DMA((2,)),
    ],
)
def _embed_kernel(
    xt_hbm, table_hbm, out_hbm,
    ibuf0, ibuf1, rows0, rows1, blk0, blk1, tmp, isem, gsem, wsem,
):
    ibuf = (ibuf0, ibuf1)
    rows = (rows0, rows1)
    blk = (blk0, blk1)

    wid = lax.axis_index("s") * NC + lax.axis_index("c")
    n0 = wid * BLOCKS_PER_W

    iota = lax.iota(jnp.int32, 16)
    jvecs = [iota * 17 + j for j in range(16)]

    def idx_off(n):
        g = n0 + n
        t = g // NBJ
        bj = g % NBJ
        return t * BATCH + bj * BBLK

    def out_base(n):
        g = n0 + n
        t = g // NBJ
        bj = g % NBJ
        return t * T_STRIDE + bj * BJ_STRIDE

    def fire_idx(n, b):
        pltpu.async_copy(
            xt_hbm.at[pl.ds(idx_off(n), BBLK)], ibuf[b], isem.at[b]
        )

    def wait_idx(n, b):
        pltpu.make_async_copy(
            xt_hbm.at[pl.ds(idx_off(n), BBLK)], ibuf[b], isem.at[b]
        ).wait()

    def fire_gather(b):
        pltpu.async_copy(table_hbm.at[ibuf[b]], rows[b], gsem.at[b])

    def wait_gather(b):
        pltpu.make_async_copy(
            table_hbm.at[ibuf[b]], rows[b], gsem.at[b]
        ).wait()

    def transpose(b):
        rows_b = rows[b]
        blk_b = blk[b]

        def cstep(cc, carry):
            cof = cc * 16 * BBLK
            for bb in range(8):
                b0 = bb * 16
                for i in range(16):
                    tmp[pl.ds(i * 17, 16)] = rows_b[b0 + i, pl.ds(cc * 16, 16)]
                for j in range(16):
                    v = plsc.load_gather(tmp, [jvecs[j]])
                    blk_b[pl.ds(cof + j * BBLK + b0, 16)] = v
            return carry

        lax.fori_loop(0, 4, cstep, 0)

    def fire_write(n, b):
        base = out_base(n)
        for ci in range(8):
            pltpu.async_copy(
                blk[b].at[pl.ds(ci * 1024, 1024)],
                out_hbm.at[pl.ds(base + ci * CI_STRIDE, 1024)],
                wsem.at[b],
            )

    def wait_write(n, b):
        base = out_base(n)
        for ci in range(8):
            pltpu.make_async_copy(
                blk[b].at[pl.ds(ci * 1024, 1024)],
                out_hbm.at[pl.ds(base + ci * CI_STRIDE, 1024)],
                wsem.at[b],
            ).wait()

    fire_idx(0, 0)
    fire_idx(1, 1)
    wait_idx(0, 0)
    fire_gather(0)

    wait_gather(0)
    transpose(0)
    fire_write(0, 0)
    fire_idx(2, 0)
    wait_idx(1, 1)
    fire_gather(1)

    wait_gather(1)
    transpose(1)
    fire_write(1, 1)
    fire_idx(3, 1)
    wait_idx(2, 0)
    fire_gather(0)

    def pair(g, carry):
        for b in range(2):
            n = g * 2 + b
            wait_gather(b)
            wait_write(n - 2, b)
            transpose(b)
            fire_write(n, b)
            fire_idx(n + 2, b)
            wait_idx(n + 1, 1 - b)
            fire_gather(1 - b)
        return carry

    lax.fori_loop(1, BLOCKS_PER_W // 2 - 1, pair, 0)

    n = BLOCKS_PER_W - 2
    wait_gather(0)
    wait_write(n - 2, 0)
    transpose(0)
    fire_write(n, 0)
    wait_idx(n + 1, 1)
    fire_gather(1)

    n = BLOCKS_PER_W - 1
    wait_gather(1)
    wait_write(n - 2, 1)
    transpose(1)
    fire_write(n, 1)

    wait_write(BLOCKS_PER_W - 2, 0)
    wait_write(BLOCKS_PER_W - 1, 1)


def kernel(x, word_table):
    xt = jnp.swapaxes(x, 0, 1).astype(jnp.int32).reshape(-1)
    flat = _embed_kernel(xt, word_table)
    out = (
        flat.reshape(HIST_LEN, 8, NBJ, 8, BBLK)
        .transpose(0, 1, 3, 2, 4)
        .reshape(HIST_LEN, EMBED_DIM, BATCH)
        .transpose(2, 0, 1)
    )
    return out
